# baseline (speedup 1.0000x reference)
import functools

import jax
import jax.numpy as jnp
from jax import lax
from jax.experimental import pallas as pl
from jax.experimental.pallas import tpu as pltpu
from jax.experimental.pallas import tpu_sc as plsc


def _mm2_body(x_ref, wr_ref, wo_ref, t_ref, r_ref):
    xb = x_ref[...]
    t_ref[...] = lax.dot_general(xb, wr_ref[...], (((1,), (1,)), ((), ())),
                                 preferred_element_type=jnp.float32)
    r_ref[...] = lax.dot_general(xb, wo_ref[...], (((1,), (1,)), ((), ())),
                                 preferred_element_type=jnp.float32)


def _mm2(x, w_rel, w_root):
    n, f = x.shape
    h = w_rel.shape[0]
    return pl.pallas_call(
        _mm2_body,
        out_shape=[
            jax.ShapeDtypeStruct((n, h), jnp.float32),
            jax.ShapeDtypeStruct((n, h), jnp.float32),
        ],
    )(x, w_rel, w_root)


def _segsum_sc(t, src2, dst2, ew2, zeros_nh):
    n, h = t.shape
    nch_all, k = src2.shape
    e = nch_all * k
    nw = 32
    nch = (e // nw) // k
    sup = 4
    nsup = nch // sup
    nq = nsup // 4
    bk = sup * k
    gpc = k // 16
    rows_per_tile = n // 16

    mesh = plsc.VectorSubcoreMesh(core_axis_name="c", subcore_axis_name="s")

    @functools.partial(
        pl.kernel,
        out_type=jax.ShapeDtypeStruct((2 * n, h), jnp.float32),
        mesh=mesh,
        compiler_params=pltpu.CompilerParams(use_tc_tiling_on_sc=False),
        scratch_types=[
            pltpu.VMEM((nch, k), jnp.int32),
            pltpu.VMEM((nch, k), jnp.int32),
            pltpu.VMEM((nch, k), jnp.float32),
            pltpu.VMEM((bk, h), jnp.float32),
            pltpu.VMEM((bk, h), jnp.float32),
            pltpu.VMEM((bk, h), jnp.float32),
            pltpu.VMEM((bk, h), jnp.float32),
            pltpu.VMEM_SHARED((n, h), jnp.float32),
            pltpu.SemaphoreType.DMA,
            pltpu.SemaphoreType.DMA,
            pltpu.SemaphoreType.DMA,
            pltpu.SemaphoreType.DMA,
            pltpu.SemaphoreType.DMA,
            pltpu.SemaphoreType.DMA,
            pltpu.SemaphoreType.DMA,
            pltpu.SemaphoreType.DMA,
        ],
    )
    def seg_kernel(t_hbm, src_hbm, dst_hbm, ew_hbm, z_hbm, out_hbm,
                   src_v, dst_v, ew_v, b0, b1, b2, b3, acc_sh,
                   g0, g1, g2, g3, s0, s1, s2, s3):
        c = lax.axis_index("c")
        s = lax.axis_index("s")
        wid = c * 16 + s
        base = wid * nch

        pltpu.sync_copy(src_hbm.at[pl.ds(base, nch)], src_v)
        pltpu.sync_copy(dst_hbm.at[pl.ds(base, nch)], dst_v)
        pltpu.sync_copy(ew_hbm.at[pl.ds(base, nch)], ew_v)

        pltpu.sync_copy(z_hbm.at[pl.ds(s * rows_per_tile, rows_per_tile)],
                        acc_sh.at[pl.ds(s * rows_per_tile, rows_per_tile)])
        plsc.subcore_barrier()

        bufs = (b0, b1, b2, b3)
        gsem = (g0, g1, g2, g3)
        ssem = (s0, s1, s2, s3)

        def fire(sidx, buf, sem):
            for cc in range(sup):
                pltpu.async_copy(
                    t_hbm.at[src_v.at[sidx * sup + cc]],
                    buf.at[pl.ds(cc * k, k)], sem)

        def drain_g(sidx, buf, sem):
            for cc in range(sup):
                pltpu.make_async_copy(
                    t_hbm.at[src_v.at[sidx * sup + cc]],
                    buf.at[pl.ds(cc * k, k)], sem).wait()

        def mult(sidx, buf):
            def group_body(g, carry):
                cc = g // gpc
                wv = ew_v[sidx * sup + cc, pl.ds((g - cc * gpc) * 16, 16)]
                for l in range(16):
                    r = g * 16 + l
                    wb = lax.gather(
                        wv, jnp.full((16, 1), l, jnp.int32),
                        lax.GatherDimensionNumbers(
                            offset_dims=(), collapsed_slice_dims=(0,),
                            start_index_map=(0,)),
                        (1,), mode=lax.GatherScatterMode.PROMISE_IN_BOUNDS)
                    buf[r, pl.ds(0, 16)] = buf[r, pl.ds(0, 16)] * wb
                    buf[r, pl.ds(16, 16)] = buf[r, pl.ds(16, 16)] * wb
                return carry

            lax.fori_loop(0, bk // 16, group_body, 0, unroll=False)

        def scat(sidx, buf, sem):
            for cc in range(sup):
                pltpu.async_copy(buf.at[pl.ds(cc * k, k)],
                                 acc_sh.at[dst_v.at[sidx * sup + cc]],
                                 sem, add=True)

        def drain_s(sidx, buf, sem):
            for cc in range(sup):
                pltpu.make_async_copy(
                    buf.at[pl.ds(cc * k, k)],
                    acc_sh.at[dst_v.at[sidx * sup + cc]], sem).wait()

        def step(s, i, first_round):
            drain_g(s, bufs[i], gsem[i])
            mult(s, bufs[i])
            scat(s, bufs[i], ssem[i])
            j = (i + 3) % 4
            if first_round:
                if i == 0:
                    fire(s + 3, bufs[j], gsem[j])
                else:
                    drain_s(s - 1, bufs[j], ssem[j])
                    fire(s + 3, bufs[j], gsem[j])
            else:
                @pl.when(s + 3 < nsup)
                def _():
                    drain_s(s - 1, bufs[j], ssem[j])
                    fire(s + 3, bufs[j], gsem[j])

        fire(0, b0, g0)
        fire(1, b1, g1)
        fire(2, b2, g2)
        for i in range(4):
            step(i, i, True)

        def round_body(q, carry):
            for i in range(4):
                step(4 * q + i, i, False)
            return carry

        lax.fori_loop(1, nq, round_body, 0, unroll=False)
        for i in range(4):
            drain_s(nsup - 4 + i, bufs[i], ssem[i])
        plsc.subcore_barrier()

        pltpu.sync_copy(
            acc_sh.at[pl.ds(s * rows_per_tile, rows_per_tile)],
            out_hbm.at[pl.ds(c * n + s * rows_per_tile, rows_per_tile)])

    return seg_kernel(t, src2, dst2, ew2, zeros_nh)


def _mid_body(agg_ref, r_ref, b_ref, wr2_ref, wo2_ref, x1_ref, t2_ref, r2_ref):
    n = r_ref.shape[0]
    a = agg_ref[pl.ds(0, n), :] + agg_ref[pl.ds(n, n), :]
    x1 = jnp.maximum(a + b_ref[...] + r_ref[...], 0.0)
    x1_ref[...] = x1
    t2_ref[...] = lax.dot_general(x1, wr2_ref[...], (((1,), (1,)), ((), ())),
                                  preferred_element_type=jnp.float32)
    r2_ref[...] = lax.dot_general(x1, wo2_ref[...], (((1,), (1,)), ((), ())),
                                  preferred_element_type=jnp.float32)


def _mid(agg2n, r1, b1, w_rel2, w_root2):
    n, h = r1.shape
    return pl.pallas_call(
        _mid_body,
        out_shape=[
            jax.ShapeDtypeStruct((n, h), jnp.float32),
            jax.ShapeDtypeStruct((n, h), jnp.float32),
            jax.ShapeDtypeStruct((n, h), jnp.float32),
        ],
    )(agg2n, r1, b1, w_rel2, w_root2)


def _head_body(agg_ref, r_ref, b_ref, x1_ref, wl_ref, bl_ref, out_ref):
    n = r_ref.shape[0]
    a = agg_ref[pl.ds(0, n), :] + agg_ref[pl.ds(n, n), :]
    x2 = jnp.maximum(a + b_ref[...] + r_ref[...], 0.0)
    hcat = jnp.concatenate([x1_ref[...], x2], axis=1)
    o = lax.dot_general(hcat, wl_ref[...], (((1,), (1,)), ((), ())),
                        preferred_element_type=jnp.float32) + bl_ref[...]
    m = jnp.max(o, axis=1, keepdims=True)
    z = o - m
    lse = jnp.log(jnp.sum(jnp.exp(z), axis=1, keepdims=True))
    out_ref[...] = z - lse


def _head(agg2n, r2, b2, x1, w_lin, b_lin):
    n, h = r2.shape
    cdim = w_lin.shape[0]
    return pl.pallas_call(
        _head_body,
        out_shape=jax.ShapeDtypeStruct((n, cdim), jnp.float32),
    )(agg2n, r2, b2, x1, w_lin, b_lin)


def kernel(x, edge_weight, W_rel1, b_rel1, W_root1, W_rel2, b_rel2, W_root2,
           W_lin, b_lin, edge_index):
    n = x.shape[0]
    h = W_rel1.shape[0]
    e = edge_weight.shape[0]
    k = 128
    quant = 32 * 4 * k
    e2 = ((e + quant - 1) // quant) * quant
    pad = e2 - e
    zi = jnp.arange(pad, dtype=jnp.int32) % n
    src2 = jnp.concatenate([edge_index[0], zi]).reshape(e2 // k, k)
    dst2 = jnp.concatenate([edge_index[1], zi]).reshape(e2 // k, k)
    ew2 = jnp.concatenate([edge_weight,
                           jnp.zeros((pad,), jnp.float32)]).reshape(e2 // k, k)
    zeros_nh = jnp.zeros((n, h), jnp.float32)

    t1, r1 = _mm2(x, W_rel1, W_root1)
    agg1 = _segsum_sc(t1, src2, dst2, ew2, zeros_nh)
    x1, t2, r2 = _mid(agg1, r1, b_rel1.reshape(1, h), W_rel2, W_root2)
    agg2 = _segsum_sc(t2, src2, dst2, ew2, zeros_nh)
    return _head(agg2, r2, b_rel2.reshape(1, h), x1, W_lin,
                 b_lin.reshape(1, -1))


# speedup vs baseline: 1.2127x; 1.2127x over previous
import functools

import jax
import jax.numpy as jnp
from jax import lax
from jax.experimental import pallas as pl
from jax.experimental.pallas import tpu as pltpu
from jax.experimental.pallas import tpu_sc as plsc


def _mm2_body(x_ref, br_ref, bo_ref, t_ref, r_ref):
    xb = x_ref[...]
    t_ref[...] = lax.dot_general(xb, br_ref[...], (((1,), (0,)), ((), ())),
                                 preferred_element_type=jnp.float32)
    r_ref[...] = lax.dot_general(xb, bo_ref[...], (((1,), (0,)), ((), ())),
                                 preferred_element_type=jnp.float32)


def _mm2(xq, b_rel, b_root):
    nq, f4 = xq.shape
    h4 = b_rel.shape[1]
    return pl.pallas_call(
        _mm2_body,
        out_shape=[
            jax.ShapeDtypeStruct((nq, h4), jnp.float32),
            jax.ShapeDtypeStruct((nq, h4), jnp.float32),
        ],
    )(xq, b_rel, b_root)


def _segsum_sc(t, src2, dst2, ew2, zeros_nh):
    n, h = t.shape
    nch_all, k = src2.shape
    e = nch_all * k
    nw = 32
    nch = (e // nw) // k
    sup = 4
    nsup = nch // sup
    nq = nsup // 4
    bk = sup * k
    gpc = k // 16
    rows_per_tile = n // 16

    mesh = plsc.VectorSubcoreMesh(core_axis_name="c", subcore_axis_name="s")

    @functools.partial(
        pl.kernel,
        out_type=jax.ShapeDtypeStruct((2 * n, h), jnp.float32),
        mesh=mesh,
        compiler_params=pltpu.CompilerParams(use_tc_tiling_on_sc=False),
        scratch_types=[
            pltpu.VMEM((nch, k), jnp.int32),
            pltpu.VMEM((nch, k), jnp.int32),
            pltpu.VMEM((nch, k), jnp.float32),
            pltpu.VMEM((bk, h), jnp.float32),
            pltpu.VMEM((bk, h), jnp.float32),
            pltpu.VMEM((bk, h), jnp.float32),
            pltpu.VMEM((bk, h), jnp.float32),
            pltpu.VMEM_SHARED((n, h), jnp.float32),
            pltpu.SemaphoreType.DMA,
            pltpu.SemaphoreType.DMA,
            pltpu.SemaphoreType.DMA,
            pltpu.SemaphoreType.DMA,
            pltpu.SemaphoreType.DMA,
            pltpu.SemaphoreType.DMA,
            pltpu.SemaphoreType.DMA,
            pltpu.SemaphoreType.DMA,
        ],
    )
    def seg_kernel(t_hbm, src_hbm, dst_hbm, ew_hbm, z_hbm, out_hbm,
                   src_v, dst_v, ew_v, b0, b1, b2, b3, acc_sh,
                   g0, g1, g2, g3, s0, s1, s2, s3):
        c = lax.axis_index("c")
        s = lax.axis_index("s")
        wid = c * 16 + s
        base = wid * nch

        pltpu.sync_copy(src_hbm.at[pl.ds(base, nch)], src_v)
        pltpu.sync_copy(dst_hbm.at[pl.ds(base, nch)], dst_v)
        pltpu.sync_copy(ew_hbm.at[pl.ds(base, nch)], ew_v)

        pltpu.sync_copy(z_hbm.at[pl.ds(s * rows_per_tile, rows_per_tile)],
                        acc_sh.at[pl.ds(s * rows_per_tile, rows_per_tile)])
        plsc.subcore_barrier()

        bufs = (b0, b1, b2, b3)
        gsem = (g0, g1, g2, g3)
        ssem = (s0, s1, s2, s3)

        def fire(sidx, buf, sem):
            for cc in range(sup):
                pltpu.async_copy(
                    t_hbm.at[src_v.at[sidx * sup + cc]],
                    buf.at[pl.ds(cc * k, k)], sem)

        def drain_g(sidx, buf, sem):
            for cc in range(sup):
                pltpu.make_async_copy(
                    t_hbm.at[src_v.at[sidx * sup + cc]],
                    buf.at[pl.ds(cc * k, k)], sem).wait()

        def mult(sidx, buf):
            def group_body(g, carry):
                cc = g // gpc
                wv = ew_v[sidx * sup + cc, pl.ds((g - cc * gpc) * 16, 16)]
                for l in range(16):
                    r = g * 16 + l
                    wb = lax.gather(
                        wv, jnp.full((16, 1), l, jnp.int32),
                        lax.GatherDimensionNumbers(
                            offset_dims=(), collapsed_slice_dims=(0,),
                            start_index_map=(0,)),
                        (1,), mode=lax.GatherScatterMode.PROMISE_IN_BOUNDS)
                    buf[r, pl.ds(0, 16)] = buf[r, pl.ds(0, 16)] * wb
                    buf[r, pl.ds(16, 16)] = buf[r, pl.ds(16, 16)] * wb
                return carry

            lax.fori_loop(0, bk // 16, group_body, 0, unroll=False)

        def scat(sidx, buf, sem):
            for cc in range(sup):
                pltpu.async_copy(buf.at[pl.ds(cc * k, k)],
                                 acc_sh.at[dst_v.at[sidx * sup + cc]],
                                 sem, add=True)

        def drain_s(sidx, buf, sem):
            for cc in range(sup):
                pltpu.make_async_copy(
                    buf.at[pl.ds(cc * k, k)],
                    acc_sh.at[dst_v.at[sidx * sup + cc]], sem).wait()

        def step(s, i, first_round):
            drain_g(s, bufs[i], gsem[i])
            mult(s, bufs[i])
            scat(s, bufs[i], ssem[i])
            j = (i + 3) % 4
            if first_round:
                if i == 0:
                    fire(s + 3, bufs[j], gsem[j])
                else:
                    drain_s(s - 1, bufs[j], ssem[j])
                    fire(s + 3, bufs[j], gsem[j])
            else:
                @pl.when(s + 3 < nsup)
                def _():
                    drain_s(s - 1, bufs[j], ssem[j])
                    fire(s + 3, bufs[j], gsem[j])

        fire(0, b0, g0)
        fire(1, b1, g1)
        fire(2, b2, g2)
        for i in range(4):
            step(i, i, True)

        def round_body(q, carry):
            for i in range(4):
                step(4 * q + i, i, False)
            return carry

        lax.fori_loop(1, nq, round_body, 0, unroll=False)
        for i in range(4):
            drain_s(nsup - 4 + i, bufs[i], ssem[i])
        plsc.subcore_barrier()

        pltpu.sync_copy(
            acc_sh.at[pl.ds(s * rows_per_tile, rows_per_tile)],
            out_hbm.at[pl.ds(c * n + s * rows_per_tile, rows_per_tile)])

    return seg_kernel(t, src2, dst2, ew2, zeros_nh)


def _mid_body(agg_ref, r_ref, b_ref, br2_ref, bo2_ref, x1_ref, t2_ref, r2_ref):
    nq = r_ref.shape[0]
    a = agg_ref[pl.ds(0, nq), :] + agg_ref[pl.ds(nq, nq), :]
    x1 = jnp.maximum(a + b_ref[...] + r_ref[...], 0.0)
    x1_ref[...] = x1
    t2_ref[...] = lax.dot_general(x1, br2_ref[...], (((1,), (0,)), ((), ())),
                                  preferred_element_type=jnp.float32)
    r2_ref[...] = lax.dot_general(x1, bo2_ref[...], (((1,), (0,)), ((), ())),
                                  preferred_element_type=jnp.float32)


def _mid(aggp, r1p, b1t, b_rel2, b_root2):
    nq, h4 = r1p.shape
    return pl.pallas_call(
        _mid_body,
        out_shape=[
            jax.ShapeDtypeStruct((nq, h4), jnp.float32),
            jax.ShapeDtypeStruct((nq, h4), jnp.float32),
            jax.ShapeDtypeStruct((nq, h4), jnp.float32),
        ],
    )(aggp, r1p, b1t, b_rel2, b_root2)


def _head_body(agg_ref, r_ref, b_ref, x1_ref, bl1_ref, bl2_ref, blb_ref,
               out_ref):
    nq = r_ref.shape[0]
    cdim = out_ref.shape[1] // 4
    a = agg_ref[pl.ds(0, nq), :] + agg_ref[pl.ds(nq, nq), :]
    x2 = jnp.maximum(a + b_ref[...] + r_ref[...], 0.0)
    o = (lax.dot_general(x1_ref[...], bl1_ref[...], (((1,), (0,)), ((), ())),
                         preferred_element_type=jnp.float32)
         + lax.dot_general(x2, bl2_ref[...], (((1,), (0,)), ((), ())),
                           preferred_element_type=jnp.float32)
         + blb_ref[...])
    for p in range(4):
        z = o[:, p * cdim:(p + 1) * cdim]
        m = jnp.max(z, axis=1, keepdims=True)
        z = z - m
        lse = jnp.log(jnp.sum(jnp.exp(z), axis=1, keepdims=True))
        out_ref[:, p * cdim:(p + 1) * cdim] = z - lse


def _head(aggp, r2p, b2t, x1p, bl1, bl2, blb):
    nq = r2p.shape[0]
    c4 = bl1.shape[1]
    return pl.pallas_call(
        _head_body,
        out_shape=jax.ShapeDtypeStruct((nq, c4), jnp.float32),
    )(aggp, r2p, b2t, x1p, bl1, bl2, blb)


def kernel(x, edge_weight, W_rel1, b_rel1, W_root1, W_rel2, b_rel2, W_root2,
           W_lin, b_lin, edge_index):
    n = x.shape[0]
    f = x.shape[1]
    h = W_rel1.shape[0]
    cdim = W_lin.shape[0]
    e = edge_weight.shape[0]
    k = 128
    quant = 32 * 4 * k
    e2 = ((e + quant - 1) // quant) * quant
    pad = e2 - e
    zi = jnp.arange(pad, dtype=jnp.int32) % n
    src2 = jnp.concatenate([edge_index[0], zi]).reshape(e2 // k, k)
    dst2 = jnp.concatenate([edge_index[1], zi]).reshape(e2 // k, k)
    ew2 = jnp.concatenate([edge_weight,
                           jnp.zeros((pad,), jnp.float32)]).reshape(e2 // k, k)
    zeros_nh = jnp.zeros((n, h), jnp.float32)

    eye4 = jnp.eye(4, dtype=jnp.float32)
    b_rel1q = jnp.kron(eye4, W_rel1.T)
    b_root1q = jnp.kron(eye4, W_root1.T)
    b_rel2q = jnp.kron(eye4, W_rel2.T)
    b_root2q = jnp.kron(eye4, W_root2.T)
    bl1q = jnp.kron(eye4, W_lin[:, :h].T)
    bl2q = jnp.kron(eye4, W_lin[:, h:].T)
    b1t = jnp.tile(b_rel1, 4).reshape(1, 4 * h)
    b2t = jnp.tile(b_rel2, 4).reshape(1, 4 * h)
    blt = jnp.tile(b_lin, 4).reshape(1, 4 * cdim)
    xq = x.reshape(n // 4, 4 * f)

    t1p, r1p = _mm2(xq, b_rel1q, b_root1q)
    agg1 = _segsum_sc(t1p.reshape(n, h), src2, dst2, ew2, zeros_nh)
    x1p, t2p, r2p = _mid(agg1.reshape(n * h // 64, 4 * h), r1p, b1t,
                         b_rel2q, b_root2q)
    agg2 = _segsum_sc(t2p.reshape(n, h), src2, dst2, ew2, zeros_nh)
    outp = _head(agg2.reshape(n * h // 64, 4 * h), r2p, b2t, x1p,
                 bl1q, bl2q, blt)
    return outp.reshape(n, cdim)
